# Initial kernel scaffold; baseline (speedup 1.0000x reference)
import jax
import jax.numpy as jnp
from jax.experimental import pallas as pl


def kernel(x_feat, params, edge_index, timesteps):
    raise NotImplementedError("write your pallas kernel here")


# speedup vs baseline: 4.1100x; 4.1100x over previous
import functools

import jax
import jax.numpy as jnp
import numpy as np
from jax import lax
from jax.experimental import pallas as pl
from jax.experimental.pallas import tpu as pltpu
from jax.experimental.pallas import tpu_sc as plsc

_CH = [16, 32, 64, 128]
_NC, _NS = 2, 16
_KB = 128
_R = 1024


@functools.cache
def _make_sc_scatter(n_pad, dout, nblk):
    mesh = plsc.VectorSubcoreMesh(core_axis_name="c", subcore_axis_name="s",
                                  num_cores=_NC, num_subcores=_NS)
    rpt = n_pad // _NS

    @functools.partial(
        pl.kernel,
        out_type=jax.ShapeDtypeStruct((_NC, n_pad, dout), jnp.float32),
        mesh=mesh,
        compiler_params=pltpu.CompilerParams(use_tc_tiling_on_sc=False, has_side_effects=True),
        scratch_types=[
            pltpu.VMEM((nblk, _KB), jnp.int32),
            pltpu.VMEM((nblk, _KB), jnp.int32),
            pltpu.VMEM((_KB, dout), jnp.float32),
            pltpu.VMEM((rpt, dout), jnp.float32),
            pltpu.VMEM_SHARED((n_pad, dout), jnp.float32),
        ],
    )
    def sc_scatter(src_hbm, dst_hbm, y_hbm, z_hbm, out_hbm,
                   src_v, dst_v, rows_v, stage_v, agg_sh):
        c = lax.axis_index("c")
        s = lax.axis_index("s")
        wid = c * _NS + s
        r0 = s * rpt
        pltpu.sync_copy(z_hbm.at[pl.ds(r0, rpt)], stage_v)
        pltpu.sync_copy(stage_v, agg_sh.at[pl.ds(r0, rpt)])
        pltpu.sync_copy(src_hbm.at[pl.ds(wid * nblk, nblk)], src_v)
        pltpu.sync_copy(dst_hbm.at[pl.ds(wid * nblk, nblk)], dst_v)
        plsc.subcore_barrier()

        def body(j, carry):
            pltpu.sync_copy(y_hbm.at[src_v.at[j]], rows_v)
            pltpu.sync_copy(rows_v, agg_sh.at[dst_v.at[j]], add=True)
            return carry

        lax.fori_loop(0, nblk, body, 0)
        plsc.subcore_barrier()
        pltpu.sync_copy(agg_sh.at[pl.ds(r0, rpt)], stage_v)
        pltpu.sync_copy(stage_v, out_hbm.at[c, pl.ds(r0, rpt)])

    return sc_scatter


def _scatter(lvl, ys):
    src2d, dst2d, n_pad, zeros, cmax = lvl
    nblk = src2d.shape[0] // (_NC * _NS)
    out = []
    for y in ys:
        cj = y.shape[1]
        k = _make_sc_scatter(n_pad, cj, nblk)
        out.append(k(src2d, dst2d, y, zeros[:, :cj]))
    return out


def _row_spec(dims, dout):
    return pl.BlockSpec((_R, dout), lambda i: (i, 0))


def _full_spec(shape):
    return pl.BlockSpec(shape, lambda i: tuple(0 for _ in shape))


def _p_specs(Ps):
    return [pl.BlockSpec((_NC, _R, P.shape[2]), lambda i: (0, i, 0))
            for P in Ps]


def _psum(prs):
    parts = [pr[0] + pr[1] for pr in prs]
    return parts[0] if len(parts) == 1 else jnp.concatenate(parts, axis=-1)


def _stage_lin(xs, ws, Ps=None, bias=None, extra=None):
    n = xs[0].shape[0]
    dout = ws[0].shape[1]
    nx = len(xs)
    np_ = len(Ps) if Ps is not None else 0
    has_b, has_e = bias is not None, extra is not None

    def body(*refs):
        i = 0
        xr = refs[i:i + nx]; i += nx
        wr = refs[i:i + nx]; i += nx
        acc = jnp.dot(xr[0][...], wr[0][...], preferred_element_type=jnp.float32)
        for t in range(1, nx):
            acc += jnp.dot(xr[t][...], wr[t][...],
                           preferred_element_type=jnp.float32)
        if np_:
            acc += _psum(refs[i:i + np_]); i += np_
        if has_b:
            acc += refs[i][...]; i += 1
        if has_e:
            acc += refs[i][...]; i += 1
        refs[i][...] = acc

    in_arrays = list(xs) + list(ws)
    in_specs = [_row_spec(n, x.shape[1]) for x in xs] + \
               [_full_spec(w.shape) for w in ws]
    if np_:
        in_arrays += list(Ps)
        in_specs += _p_specs(Ps)
    if has_b:
        in_arrays.append(bias.reshape(1, dout))
        in_specs.append(_full_spec((1, dout)))
    if has_e:
        in_arrays.append(extra)
        in_specs.append(_row_spec(n, dout))
    return pl.pallas_call(
        body,
        grid=(pl.cdiv(n, _R),),
        in_specs=in_specs,
        out_specs=_row_spec(n, dout),
        out_shape=jax.ShapeDtypeStruct((n, dout), jnp.float32),
    )(*in_arrays)


def _chunk_widths(dout, cmax):
    if dout <= cmax:
        return [dout]
    assert dout % cmax == 0
    return [cmax] * (dout // cmax)


def _stage_y(xs, ws, n_pad, chunks):
    n = xs[0].shape[0]
    nx = len(xs)

    def body(*refs):
        i = 0
        xr = refs[i:i + nx]; i += nx
        wr = refs[i:i + nx]; i += nx
        acc = jnp.dot(xr[0][...], wr[0][...], preferred_element_type=jnp.float32)
        for t in range(1, nx):
            acc += jnp.dot(xr[t][...], wr[t][...],
                           preferred_element_type=jnp.float32)
        o = 0
        for j, cj in enumerate(chunks):
            refs[i + j][...] = acc[:, o:o + cj]
            o += cj

    in_specs = [_row_spec(n, x.shape[1]) for x in xs] + \
               [_full_spec(w.shape) for w in ws]
    out = pl.pallas_call(
        body,
        grid=(pl.cdiv(n_pad, _R),),
        in_specs=in_specs,
        out_specs=[_row_spec(n_pad, cj) for cj in chunks],
        out_shape=[jax.ShapeDtypeStruct((n_pad, cj), jnp.float32)
                   for cj in chunks],
    )(*xs, *ws)
    return list(out) if isinstance(out, (list, tuple)) else [out]


def _stage_res_mid(xs, ws, Ps, bias, w2n, wsk, n_pad, chunks):
    n = xs[0].shape[0]
    dout = ws[0].shape[1]
    dsk = wsk[0].shape[1]
    nx = len(xs)
    np_ = len(Ps)

    def body(*refs):
        i = 0
        xr = refs[i:i + nx]; i += nx
        wr = refs[i:i + nx]; i += nx
        prs = refs[i:i + np_]; i += np_
        br = refs[i]; i += 1
        w2r = refs[i]; i += 1
        skr = refs[i:i + nx]; i += nx
        acc = jnp.dot(xr[0][...], wr[0][...], preferred_element_type=jnp.float32)
        sk = jnp.dot(xr[0][...], skr[0][...], preferred_element_type=jnp.float32)
        for t in range(1, nx):
            acc += jnp.dot(xr[t][...], wr[t][...],
                           preferred_element_type=jnp.float32)
            sk += jnp.dot(xr[t][...], skr[t][...],
                          preferred_element_type=jnp.float32)
        acc = acc + _psum(prs) + br[...]
        h = acc * jax.nn.sigmoid(acc)
        y2 = jnp.dot(h, w2r[...], preferred_element_type=jnp.float32)
        refs[i][...] = h; i += 1
        o = 0
        for cj in chunks:
            refs[i][...] = y2[:, o:o + cj]; i += 1; o += cj
        refs[i][...] = sk

    in_arrays = list(xs) + list(ws) + list(Ps) + \
        [bias.reshape(1, dout), w2n] + list(wsk)
    in_specs = [_row_spec(n, x.shape[1]) for x in xs] + \
               [_full_spec(w.shape) for w in ws] + _p_specs(Ps) + \
               [_full_spec((1, dout)), _full_spec(w2n.shape)] + \
               [_full_spec(w.shape) for w in wsk]
    outs = pl.pallas_call(
        body,
        grid=(pl.cdiv(n_pad, _R),),
        in_specs=in_specs,
        out_specs=[_row_spec(n_pad, dout)] +
                  [_row_spec(n_pad, cj) for cj in chunks] +
                  [_row_spec(n_pad, dsk)],
        out_shape=[jax.ShapeDtypeStruct((n_pad, dout), jnp.float32)] +
                  [jax.ShapeDtypeStruct((n_pad, cj), jnp.float32)
                   for cj in chunks] +
                  [jax.ShapeDtypeStruct((n_pad, dsk), jnp.float32)],
    )(*in_arrays)
    return outs[0], outs[1:-1], outs[-1]


def _pool(x):
    n, ch = x.shape
    x2 = x.reshape(n // 2, 2 * ch)

    def body(x_ref, o_ref):
        o_ref[...] = 0.5 * (x_ref[:, :ch] + x_ref[:, ch:])

    return pl.pallas_call(
        body,
        grid=(pl.cdiv(n // 2, _R),),
        in_specs=[_row_spec(n // 2, 2 * ch)],
        out_specs=_row_spec(n // 2, ch),
        out_shape=jax.ShapeDtypeStruct((n // 2, ch), jnp.float32),
    )(x2)


def _unpool(x):
    nc, ch = x.shape

    def body(x_ref, o_ref):
        v = x_ref[...]
        o_ref[:, :ch] = v
        o_ref[:, ch:] = v

    out = pl.pallas_call(
        body,
        grid=(pl.cdiv(nc, _R),),
        in_specs=[_row_spec(nc, ch)],
        out_specs=_row_spec(nc, 2 * ch),
        out_shape=jax.ShapeDtypeStruct((nc, 2 * ch), jnp.float32),
    )(x)
    return out.reshape(2 * nc, ch)


def _temb_stage(t_emb, w1, b1, w2, b2, wcat, bcat):
    s = wcat.shape[1]

    def body(t_ref, w1r, b1r, w2r, b2r, wcr, bcr, o_ref):
        z = jnp.dot(t_ref[...], w1r[...], preferred_element_type=jnp.float32) \
            + b1r[...]
        z = z * jax.nn.sigmoid(z)
        z = jnp.dot(z, w2r[...], preferred_element_type=jnp.float32) + b2r[...]
        z = z * jax.nn.sigmoid(z)
        o_ref[...] = jnp.dot(z, wcr[...], preferred_element_type=jnp.float32) \
            + bcr[...]

    args = [t_emb, w1, b1.reshape(1, -1), w2, b2.reshape(1, -1),
            wcat, bcat.reshape(1, -1)]
    return pl.pallas_call(
        body,
        grid=(1,),
        in_specs=[_full_spec(a.shape) for a in args],
        out_specs=_full_spec((1, s)),
        out_shape=jax.ShapeDtypeStruct((1, s), jnp.float32),
    )(*args)


def _t_embedding(t, dim, max_period=10000.0):
    half = dim // 2
    exponent = -np.log(max_period) * jnp.arange(half, dtype=jnp.float32) / half
    freqs = jnp.exp(exponent)
    args = jnp.float32(t) * freqs
    emb = jnp.concatenate([jnp.sin(args), jnp.cos(args)])
    emb = jnp.concatenate([emb[half:], emb[:half]])
    return emb[None, :]


def _split_rows(w, sizes):
    out, o = [], 0
    for sz in sizes:
        out.append(w[o:o + sz]); o += sz
    return out


def _res_block(p, xs, lvl, tvec):
    src2d, dst2d, n_pad, zeros, cmax = lvl
    sizes = [x.shape[1] for x in xs]
    dout = p["c1"]["wn"].shape[1]
    chunks = _chunk_widths(dout, cmax)
    y1s = _stage_y(xs, _split_rows(p["c1"]["wn"], sizes), n_pad, chunks)
    P1s = _scatter(lvl, y1s)
    h, y2s, sk = _stage_res_mid(xs, _split_rows(p["c1"]["ws"], sizes), P1s,
                                p["c1"]["b"] + tvec, p["c2"]["wn"],
                                _split_rows(p["skip"]["w"], sizes),
                                n_pad, chunks)
    P2s = _scatter(lvl, y2s)
    return _stage_lin([h], [p["c2"]["ws"]], P2s,
                      p["c2"]["b"] + p["skip"]["b"], extra=sk)


def kernel(x_feat, params, edge_index, timesteps):
    n0 = x_feat.shape[0]
    ns = [n0, n0 // 2, n0 // 4, n0 // 8]
    E = edge_index.shape[1]
    nblk = -(-(-(-E // (_KB * _NC * _NS))) // 8) * 8
    e_pad = nblk * _KB * _NC * _NS

    src0 = edge_index[0]
    dst0 = edge_index[1]
    pad = e_pad - E
    levels = []
    n_pad0 = 1 << (ns[0] - 1).bit_length()
    for l, nl in enumerate(ns):
        n_pad = n_pad0 >> l
        cmax = min(128, 524288 // n_pad)
        s = jnp.concatenate([src0 >> l, jnp.zeros((pad,), jnp.int32)])
        d = jnp.concatenate([dst0 >> l, jnp.full((pad,), nl, jnp.int32)])
        zeros = jnp.zeros((n_pad, cmax), jnp.float32)
        levels.append((s.reshape(-1, _KB), d.reshape(-1, _KB), n_pad, zeros,
                       cmax))
    L0, L1, L2, L3 = levels

    blocks = ["b1_0", "b1_1", "b2_0", "b2_1", "b3_0", "b3_1", "mid_0",
              "mid_1", "u3_0", "u3_1", "u3_2", "u2_0", "u2_1", "u2_2",
              "u1_0", "u1_1", "u1_2"]
    wcat = jnp.concatenate([params[b]["t"]["w"] for b in blocks], axis=1)
    bcat = jnp.concatenate([params[b]["t"]["b"] for b in blocks])
    t_emb = _t_embedding(timesteps, _CH[0])
    tcat = _temb_stage(t_emb, params["temb1"]["w"], params["temb1"]["b"],
                       params["temb2"]["w"], params["temb2"]["b"], wcat, bcat)[0]
    tvecs, o = {}, 0
    for b in blocks:
        d = params[b]["t"]["w"].shape[1]
        tvecs[b] = tcat[o:o + d]; o += d

    xf8 = jnp.pad(x_feat, ((0, L0[2] - n0, ), (0, 4)))
    pad4 = lambda w: jnp.pad(w, ((0, 4), (0, 0)))
    wn35 = pad4(jnp.concatenate([params["sp3"]["wn"], params["sp5"]["wn"]],
                                axis=1))
    ws35 = pad4(jnp.concatenate([params["sp3"]["ws"], params["sp5"]["ws"]],
                                axis=1))
    b35 = jnp.concatenate([params["sp3"]["b"], params["sp5"]["b"]])
    npd0, cmax0 = L0[2], L0[4]
    y35 = _stage_y([xf8], [wn35], npd0, _chunk_widths(8, cmax0))
    o35 = _stage_lin([xf8], [ws35], _scatter(L0, y35), b35)

    ci = params["conv_in"]
    wn_ci = [pad4(ci["wn"][:4]), ci["wn"][4:]]
    ws_ci = [pad4(ci["ws"][:4]), ci["ws"][4:]]
    y_ci = _stage_y([xf8, o35], wn_ci, npd0, _chunk_widths(16, cmax0))
    x0 = _stage_lin([xf8, o35], ws_ci, _scatter(L0, y_ci), ci["b"])

    h = _res_block(params["b1_0"], [x0], L0, tvecs["b1_0"])
    out_s1 = _res_block(params["b1_1"], [_pool(h)], L1, tvecs["b1_1"])
    h = _res_block(params["b2_0"], [out_s1], L1, tvecs["b2_0"])
    out_s2 = _res_block(params["b2_1"], [_pool(h)], L2, tvecs["b2_1"])
    h = _res_block(params["b3_0"], [out_s2], L2, tvecs["b3_0"])
    out_s3 = _res_block(params["b3_1"], [_pool(h)], L3, tvecs["b3_1"])
    out = _res_block(params["mid_0"], [out_s3], L3, tvecs["mid_0"])
    out = _res_block(params["mid_1"], [out], L3, tvecs["mid_1"])
    out = _unpool(_res_block(params["u3_0"], [out], L3, tvecs["u3_0"]))
    out = _res_block(params["u3_1"], [out], L2, tvecs["u3_1"])
    out = _res_block(params["u3_2"], [out], L2, tvecs["u3_2"])
    out = _unpool(_res_block(params["u2_0"], [out, out_s2], L2, tvecs["u2_0"]))
    out = _res_block(params["u2_1"], [out], L1, tvecs["u2_1"])
    out = _res_block(params["u2_2"], [out], L1, tvecs["u2_2"])
    out = _unpool(_res_block(params["u1_0"], [out, out_s1], L1, tvecs["u1_0"]))
    out = _res_block(params["u1_1"], [out], L0, tvecs["u1_1"])
    out = _res_block(params["u1_2"], [out], L0, tvecs["u1_2"])

    co = params["conv_out"]
    wn_co = jnp.pad(co["wn"], ((0, 0), (0, 4)))
    ws_co = jnp.pad(co["ws"], ((0, 0), (0, 4)))
    b_co = jnp.pad(co["b"], (0, 4))
    y = _stage_y([out], [wn_co], npd0, [8])
    out8 = _stage_lin([out], [ws_co], _scatter(L0, y), b_co)
    return out8[:n0, :4]


# speedup vs baseline: 5.2035x; 1.2661x over previous
import functools

import jax
import jax.numpy as jnp
import numpy as np
from jax import lax
from jax.experimental import pallas as pl
from jax.experimental.pallas import tpu as pltpu
from jax.experimental.pallas import tpu_sc as plsc

_CH = [16, 32, 64, 128]
_NC, _NS = 2, 16
_KB = 128
_R = 1024


@functools.cache
def _make_sc_scatter(n_pad, dout, nblk):
    mesh = plsc.VectorSubcoreMesh(core_axis_name="c", subcore_axis_name="s",
                                  num_cores=_NC, num_subcores=_NS)
    rpt = n_pad // _NS

    @functools.partial(
        pl.kernel,
        out_type=jax.ShapeDtypeStruct((_NC, n_pad, dout), jnp.float32),
        mesh=mesh,
        compiler_params=pltpu.CompilerParams(use_tc_tiling_on_sc=False, has_side_effects=True),
        scratch_types=[
            pltpu.VMEM((nblk, _KB), jnp.int32),
            pltpu.VMEM((nblk, _KB), jnp.int32),
            pltpu.VMEM((_KB, dout), jnp.float32),
            pltpu.VMEM((_KB, dout), jnp.float32),
            pltpu.VMEM((_KB, dout), jnp.float32),
            pltpu.VMEM((_KB, dout), jnp.float32),
            pltpu.VMEM((rpt // 2, dout), jnp.float32),
            pltpu.SemaphoreType.DMA,
            pltpu.SemaphoreType.DMA,
            pltpu.SemaphoreType.DMA,
            pltpu.SemaphoreType.DMA,
            pltpu.VMEM_SHARED((n_pad, dout), jnp.float32),
        ],
    )
    def sc_scatter(src_hbm, dst_hbm, y_hbm, z_hbm, out_hbm,
                   src_v, dst_v, r0v, r1v, r2v, r3v, stage_v,
                   g0, g1, g2, g3, agg_sh):
        rows = (r0v, r1v, r2v, r3v)
        gsem = (g0, g1, g2, g3)
        c = lax.axis_index("c")
        s = lax.axis_index("s")
        wid = c * _NS + s
        r0 = s * rpt
        hr = rpt // 2
        for q in range(2):
            pltpu.sync_copy(z_hbm.at[pl.ds(r0 + q * hr, hr)], stage_v)
            pltpu.sync_copy(stage_v, agg_sh.at[pl.ds(r0 + q * hr, hr)])
        pltpu.sync_copy(src_hbm.at[pl.ds(wid * nblk, nblk)], src_v)
        pltpu.sync_copy(dst_hbm.at[pl.ds(wid * nblk, nblk)], dst_v)
        plsc.subcore_barrier()

        for b in range(4):
            pltpu.async_copy(y_hbm.at[src_v.at[b]], rows[b], gsem[b])

        def body(i, carry):
            j = i * 4
            for b in range(4):
                pltpu.make_async_copy(y_hbm.at[src_v.at[j + b]],
                                      rows[b], gsem[b]).wait()
                pltpu.sync_copy(rows[b], agg_sh.at[dst_v.at[j + b]],
                                add=True)

                @pl.when(j + b + 4 < nblk)
                def _():
                    pltpu.async_copy(y_hbm.at[src_v.at[j + b + 4]],
                                     rows[b], gsem[b])
            return carry

        lax.fori_loop(0, nblk // 4, body, 0)
        plsc.subcore_barrier()
        for q in range(2):
            pltpu.sync_copy(agg_sh.at[pl.ds(r0 + q * hr, hr)], stage_v)
            pltpu.sync_copy(stage_v, out_hbm.at[c, pl.ds(r0 + q * hr, hr)])

    return sc_scatter


def _scatter(lvl, ys):
    src2d, dst2d, n_pad, zeros, cmax = lvl
    nblk = src2d.shape[0] // (_NC * _NS)
    out = []
    for y in ys:
        cj = y.shape[1]
        k = _make_sc_scatter(n_pad, cj, nblk)
        out.append(k(src2d, dst2d, y, zeros[:, :cj]))
    return out


def _row_spec(dims, dout):
    return pl.BlockSpec((_R, dout), lambda i: (i, 0))


def _full_spec(shape):
    return pl.BlockSpec(shape, lambda i: tuple(0 for _ in shape))


def _p_specs(Ps):
    return [pl.BlockSpec((_NC, _R, P.shape[2]), lambda i: (0, i, 0))
            for P in Ps]


def _psum(prs):
    parts = [pr[0] + pr[1] for pr in prs]
    return parts[0] if len(parts) == 1 else jnp.concatenate(parts, axis=-1)


def _stage_lin(xs, ws, Ps=None, bias=None, extra=None):
    n = xs[0].shape[0]
    dout = ws[0].shape[1]
    nx = len(xs)
    np_ = len(Ps) if Ps is not None else 0
    has_b, has_e = bias is not None, extra is not None

    def body(*refs):
        i = 0
        xr = refs[i:i + nx]; i += nx
        wr = refs[i:i + nx]; i += nx
        acc = jnp.dot(xr[0][...], wr[0][...], preferred_element_type=jnp.float32)
        for t in range(1, nx):
            acc += jnp.dot(xr[t][...], wr[t][...],
                           preferred_element_type=jnp.float32)
        if np_:
            acc += _psum(refs[i:i + np_]); i += np_
        if has_b:
            acc += refs[i][...]; i += 1
        if has_e:
            acc += refs[i][...]; i += 1
        refs[i][...] = acc

    in_arrays = list(xs) + list(ws)
    in_specs = [_row_spec(n, x.shape[1]) for x in xs] + \
               [_full_spec(w.shape) for w in ws]
    if np_:
        in_arrays += list(Ps)
        in_specs += _p_specs(Ps)
    if has_b:
        in_arrays.append(bias.reshape(1, dout))
        in_specs.append(_full_spec((1, dout)))
    if has_e:
        in_arrays.append(extra)
        in_specs.append(_row_spec(n, dout))
    return pl.pallas_call(
        body,
        grid=(pl.cdiv(n, _R),),
        in_specs=in_specs,
        out_specs=_row_spec(n, dout),
        out_shape=jax.ShapeDtypeStruct((n, dout), jnp.float32),
    )(*in_arrays)


def _chunk_widths(dout, cmax):
    if dout <= cmax:
        return [dout]
    assert dout % cmax == 0
    return [cmax] * (dout // cmax)


def _stage_y(xs, ws, n_pad, chunks):
    n = xs[0].shape[0]
    nx = len(xs)

    def body(*refs):
        i = 0
        xr = refs[i:i + nx]; i += nx
        wr = refs[i:i + nx]; i += nx
        acc = jnp.dot(xr[0][...], wr[0][...], preferred_element_type=jnp.float32)
        for t in range(1, nx):
            acc += jnp.dot(xr[t][...], wr[t][...],
                           preferred_element_type=jnp.float32)
        o = 0
        for j, cj in enumerate(chunks):
            refs[i + j][...] = acc[:, o:o + cj]
            o += cj

    in_specs = [_row_spec(n, x.shape[1]) for x in xs] + \
               [_full_spec(w.shape) for w in ws]
    out = pl.pallas_call(
        body,
        grid=(pl.cdiv(n_pad, _R),),
        in_specs=in_specs,
        out_specs=[_row_spec(n_pad, cj) for cj in chunks],
        out_shape=[jax.ShapeDtypeStruct((n_pad, cj), jnp.float32)
                   for cj in chunks],
    )(*xs, *ws)
    return list(out) if isinstance(out, (list, tuple)) else [out]


def _stage_res_mid(xs, ws, Ps, bias, w2n, wsk, n_pad, chunks):
    n = xs[0].shape[0]
    dout = ws[0].shape[1]
    dsk = wsk[0].shape[1]
    nx = len(xs)
    np_ = len(Ps)

    def body(*refs):
        i = 0
        xr = refs[i:i + nx]; i += nx
        wr = refs[i:i + nx]; i += nx
        prs = refs[i:i + np_]; i += np_
        br = refs[i]; i += 1
        w2r = refs[i]; i += 1
        skr = refs[i:i + nx]; i += nx
        acc = jnp.dot(xr[0][...], wr[0][...], preferred_element_type=jnp.float32)
        sk = jnp.dot(xr[0][...], skr[0][...], preferred_element_type=jnp.float32)
        for t in range(1, nx):
            acc += jnp.dot(xr[t][...], wr[t][...],
                           preferred_element_type=jnp.float32)
            sk += jnp.dot(xr[t][...], skr[t][...],
                          preferred_element_type=jnp.float32)
        acc = acc + _psum(prs) + br[...]
        h = acc * jax.nn.sigmoid(acc)
        y2 = jnp.dot(h, w2r[...], preferred_element_type=jnp.float32)
        refs[i][...] = h; i += 1
        o = 0
        for cj in chunks:
            refs[i][...] = y2[:, o:o + cj]; i += 1; o += cj
        refs[i][...] = sk

    in_arrays = list(xs) + list(ws) + list(Ps) + \
        [bias.reshape(1, dout), w2n] + list(wsk)
    in_specs = [_row_spec(n, x.shape[1]) for x in xs] + \
               [_full_spec(w.shape) for w in ws] + _p_specs(Ps) + \
               [_full_spec((1, dout)), _full_spec(w2n.shape)] + \
               [_full_spec(w.shape) for w in wsk]
    outs = pl.pallas_call(
        body,
        grid=(pl.cdiv(n_pad, _R),),
        in_specs=in_specs,
        out_specs=[_row_spec(n_pad, dout)] +
                  [_row_spec(n_pad, cj) for cj in chunks] +
                  [_row_spec(n_pad, dsk)],
        out_shape=[jax.ShapeDtypeStruct((n_pad, dout), jnp.float32)] +
                  [jax.ShapeDtypeStruct((n_pad, cj), jnp.float32)
                   for cj in chunks] +
                  [jax.ShapeDtypeStruct((n_pad, dsk), jnp.float32)],
    )(*in_arrays)
    return outs[0], outs[1:-1], outs[-1]


def _pool(x):
    n, ch = x.shape
    x2 = x.reshape(n // 2, 2 * ch)

    def body(x_ref, o_ref):
        o_ref[...] = 0.5 * (x_ref[:, :ch] + x_ref[:, ch:])

    return pl.pallas_call(
        body,
        grid=(pl.cdiv(n // 2, _R),),
        in_specs=[_row_spec(n // 2, 2 * ch)],
        out_specs=_row_spec(n // 2, ch),
        out_shape=jax.ShapeDtypeStruct((n // 2, ch), jnp.float32),
    )(x2)


def _unpool(x):
    nc, ch = x.shape

    def body(x_ref, o_ref):
        v = x_ref[...]
        o_ref[:, :ch] = v
        o_ref[:, ch:] = v

    out = pl.pallas_call(
        body,
        grid=(pl.cdiv(nc, _R),),
        in_specs=[_row_spec(nc, ch)],
        out_specs=_row_spec(nc, 2 * ch),
        out_shape=jax.ShapeDtypeStruct((nc, 2 * ch), jnp.float32),
    )(x)
    return out.reshape(2 * nc, ch)


def _temb_stage(t_emb, w1, b1, w2, b2, wcat, bcat):
    s = wcat.shape[1]

    def body(t_ref, w1r, b1r, w2r, b2r, wcr, bcr, o_ref):
        z = jnp.dot(t_ref[...], w1r[...], preferred_element_type=jnp.float32) \
            + b1r[...]
        z = z * jax.nn.sigmoid(z)
        z = jnp.dot(z, w2r[...], preferred_element_type=jnp.float32) + b2r[...]
        z = z * jax.nn.sigmoid(z)
        o_ref[...] = jnp.dot(z, wcr[...], preferred_element_type=jnp.float32) \
            + bcr[...]

    args = [t_emb, w1, b1.reshape(1, -1), w2, b2.reshape(1, -1),
            wcat, bcat.reshape(1, -1)]
    return pl.pallas_call(
        body,
        grid=(1,),
        in_specs=[_full_spec(a.shape) for a in args],
        out_specs=_full_spec((1, s)),
        out_shape=jax.ShapeDtypeStruct((1, s), jnp.float32),
    )(*args)


def _t_embedding(t, dim, max_period=10000.0):
    half = dim // 2
    exponent = -np.log(max_period) * jnp.arange(half, dtype=jnp.float32) / half
    freqs = jnp.exp(exponent)
    args = jnp.float32(t) * freqs
    emb = jnp.concatenate([jnp.sin(args), jnp.cos(args)])
    emb = jnp.concatenate([emb[half:], emb[:half]])
    return emb[None, :]


def _split_rows(w, sizes):
    out, o = [], 0
    for sz in sizes:
        out.append(w[o:o + sz]); o += sz
    return out


def _res_block(p, xs, lvl, tvec):
    src2d, dst2d, n_pad, zeros, cmax = lvl
    sizes = [x.shape[1] for x in xs]
    dout = p["c1"]["wn"].shape[1]
    chunks = _chunk_widths(dout, cmax)
    y1s = _stage_y(xs, _split_rows(p["c1"]["wn"], sizes), n_pad, chunks)
    P1s = _scatter(lvl, y1s)
    h, y2s, sk = _stage_res_mid(xs, _split_rows(p["c1"]["ws"], sizes), P1s,
                                p["c1"]["b"] + tvec, p["c2"]["wn"],
                                _split_rows(p["skip"]["w"], sizes),
                                n_pad, chunks)
    P2s = _scatter(lvl, y2s)
    return _stage_lin([h], [p["c2"]["ws"]], P2s,
                      p["c2"]["b"] + p["skip"]["b"], extra=sk)


def kernel(x_feat, params, edge_index, timesteps):
    n0 = x_feat.shape[0]
    ns = [n0, n0 // 2, n0 // 4, n0 // 8]
    E = edge_index.shape[1]
    nblk = -(-(-(-E // (_KB * _NC * _NS))) // 8) * 8
    e_pad = nblk * _KB * _NC * _NS

    src0 = edge_index[0]
    dst0 = edge_index[1]
    pad = e_pad - E
    levels = []
    n_pad0 = 1 << (ns[0] - 1).bit_length()
    for l, nl in enumerate(ns):
        n_pad = n_pad0 >> l
        cmax = min(32, 524288 // n_pad)
        s = jnp.concatenate([src0 >> l, jnp.zeros((pad,), jnp.int32)])
        d = jnp.concatenate([dst0 >> l, jnp.full((pad,), nl, jnp.int32)])
        zeros = jnp.zeros((n_pad, cmax), jnp.float32)
        levels.append((s.reshape(-1, _KB), d.reshape(-1, _KB), n_pad, zeros,
                       cmax))
    L0, L1, L2, L3 = levels

    blocks = ["b1_0", "b1_1", "b2_0", "b2_1", "b3_0", "b3_1", "mid_0",
              "mid_1", "u3_0", "u3_1", "u3_2", "u2_0", "u2_1", "u2_2",
              "u1_0", "u1_1", "u1_2"]
    wcat = jnp.concatenate([params[b]["t"]["w"] for b in blocks], axis=1)
    bcat = jnp.concatenate([params[b]["t"]["b"] for b in blocks])
    t_emb = _t_embedding(timesteps, _CH[0])
    tcat = _temb_stage(t_emb, params["temb1"]["w"], params["temb1"]["b"],
                       params["temb2"]["w"], params["temb2"]["b"], wcat, bcat)[0]
    tvecs, o = {}, 0
    for b in blocks:
        d = params[b]["t"]["w"].shape[1]
        tvecs[b] = tcat[o:o + d]; o += d

    xf8 = jnp.pad(x_feat, ((0, L0[2] - n0, ), (0, 4)))
    pad4 = lambda w: jnp.pad(w, ((0, 4), (0, 0)))
    wn35 = pad4(jnp.concatenate([params["sp3"]["wn"], params["sp5"]["wn"]],
                                axis=1))
    ws35 = pad4(jnp.concatenate([params["sp3"]["ws"], params["sp5"]["ws"]],
                                axis=1))
    b35 = jnp.concatenate([params["sp3"]["b"], params["sp5"]["b"]])
    npd0, cmax0 = L0[2], L0[4]
    y35 = _stage_y([xf8], [wn35], npd0, _chunk_widths(8, cmax0))
    o35 = _stage_lin([xf8], [ws35], _scatter(L0, y35), b35)

    ci = params["conv_in"]
    wn_ci = [pad4(ci["wn"][:4]), ci["wn"][4:]]
    ws_ci = [pad4(ci["ws"][:4]), ci["ws"][4:]]
    y_ci = _stage_y([xf8, o35], wn_ci, npd0, _chunk_widths(16, cmax0))
    x0 = _stage_lin([xf8, o35], ws_ci, _scatter(L0, y_ci), ci["b"])

    h = _res_block(params["b1_0"], [x0], L0, tvecs["b1_0"])
    out_s1 = _res_block(params["b1_1"], [_pool(h)], L1, tvecs["b1_1"])
    h = _res_block(params["b2_0"], [out_s1], L1, tvecs["b2_0"])
    out_s2 = _res_block(params["b2_1"], [_pool(h)], L2, tvecs["b2_1"])
    h = _res_block(params["b3_0"], [out_s2], L2, tvecs["b3_0"])
    out_s3 = _res_block(params["b3_1"], [_pool(h)], L3, tvecs["b3_1"])
    out = _res_block(params["mid_0"], [out_s3], L3, tvecs["mid_0"])
    out = _res_block(params["mid_1"], [out], L3, tvecs["mid_1"])
    out = _unpool(_res_block(params["u3_0"], [out], L3, tvecs["u3_0"]))
    out = _res_block(params["u3_1"], [out], L2, tvecs["u3_1"])
    out = _res_block(params["u3_2"], [out], L2, tvecs["u3_2"])
    out = _unpool(_res_block(params["u2_0"], [out, out_s2], L2, tvecs["u2_0"]))
    out = _res_block(params["u2_1"], [out], L1, tvecs["u2_1"])
    out = _res_block(params["u2_2"], [out], L1, tvecs["u2_2"])
    out = _unpool(_res_block(params["u1_0"], [out, out_s1], L1, tvecs["u1_0"]))
    out = _res_block(params["u1_1"], [out], L0, tvecs["u1_1"])
    out = _res_block(params["u1_2"], [out], L0, tvecs["u1_2"])

    co = params["conv_out"]
    wn_co = jnp.pad(co["wn"], ((0, 0), (0, 4)))
    ws_co = jnp.pad(co["ws"], ((0, 0), (0, 4)))
    b_co = jnp.pad(co["b"], (0, 4))
    y = _stage_y([out], [wn_co], npd0, [8])
    out8 = _stage_lin([out], [ws_co], _scatter(L0, y), b_co)
    return out8[:n0, :4]
